# baseline (speedup 1.0000x reference)
import functools

import jax
import jax.numpy as jnp
from jax import lax
from jax.experimental import pallas as pl
from jax.experimental.pallas import tpu as pltpu
from jax.experimental.pallas import tpu_sc as plsc

B = 4096
L = 200
D = 32
N = B * L
NC = 2
NS = 16
NW = NC * NS
BPW = B // NW
CB = 8
NCH = BPW // CB
R = CB * L
LANES = 16
NBUF = 2


def _fire(x_hbm, tab_hbm, idx_v, rows_v, sems, g, b0):
    buf = g % NBUF
    bc = b0 + g * CB
    pltpu.sync_copy(x_hbm.at[pl.ds(bc * L, R)], idx_v.at[buf])
    for j in range(CB):
        pltpu.async_copy(
            tab_hbm.at[idx_v.at[buf, pl.ds(j * L, L)]],
            rows_v.at[buf, j], sems.at[buf])


def _drain(tab_hbm, idx_v, rows_v, sems, g):
    buf = g % NBUF
    for j in range(CB):
        pltpu.make_async_copy(
            tab_hbm.at[idx_v.at[buf, pl.ds(j * L, L)]],
            rows_v.at[buf, j], sems.at[buf]).wait()


def _emb_body(x_hbm, tab_hbm, pos_hbm, out_hbm, idx_v, rows_v, pos_v, sems):
    wid = lax.axis_index("s") * NC + lax.axis_index("c")
    b0 = wid * BPW

    pltpu.sync_copy(pos_hbm, pos_v)

    _fire(x_hbm, tab_hbm, idx_v, rows_v, sems, 0, b0)

    def chunk_body(g, carry):
        buf = g % NBUF

        @pl.when(g + 1 < NCH)
        def _():
            _fire(x_hbm, tab_hbm, idx_v, rows_v, sems, g + 1, b0)

        _drain(tab_hbm, idx_v, rows_v, sems, g)

        def pos_body(l, c1):
            p0 = pos_v[l, pl.ds(0, LANES)]
            p1 = pos_v[l, pl.ds(LANES, LANES)]
            for j in range(CB):
                rows_v[buf, j, l, pl.ds(0, LANES)] = (
                    rows_v[buf, j, l, pl.ds(0, LANES)] + p0)
                rows_v[buf, j, l, pl.ds(LANES, LANES)] = (
                    rows_v[buf, j, l, pl.ds(LANES, LANES)] + p1)
            return c1
        lax.fori_loop(0, L, pos_body, 0)

        bc = b0 + g * CB
        pltpu.sync_copy(rows_v.at[buf], out_hbm.at[pl.ds(bc, CB)])
        return carry

    lax.fori_loop(0, NCH, chunk_body, 0)


@jax.jit
def _emb(x_flat, table, pos):
    mesh = plsc.VectorSubcoreMesh(core_axis_name="c", subcore_axis_name="s")
    return pl.kernel(
        _emb_body,
        out_type=jax.ShapeDtypeStruct((B, L, D), jnp.float32),
        mesh=mesh,
        compiler_params=pltpu.CompilerParams(use_tc_tiling_on_sc=False),
        scratch_types=[
            pltpu.VMEM((NBUF, R), jnp.int32),
            pltpu.VMEM((NBUF, CB, L, D), jnp.float32),
            pltpu.VMEM((L, D), jnp.float32),
            pltpu.SemaphoreType.DMA((NBUF,)),
        ],
    )(x_flat, table, pos)


def kernel(x, embedding_table, possitional_emb):
    return _emb(x.reshape(-1).astype(jnp.int32), embedding_table,
                possitional_emb)
